# Initial kernel scaffold; baseline (speedup 1.0000x reference)
import jax
import jax.numpy as jnp
from jax.experimental import pallas as pl


def kernel(X, vertex, edges, W, att_e):
    raise NotImplementedError("write your pallas kernel here")


# speedup vs baseline: 74.5901x; 74.5901x over previous
import functools

import jax
import jax.numpy as jnp
from jax import lax
from jax.experimental import pallas as pl
from jax.experimental.pallas import tpu as pltpu
from jax.experimental.pallas import tpu_sc as plsc

_N = 10000
_E = 20000
_NNZ = 320000
_H = 4
_C = 32
_HC = _H * _C
_HALF = _HC // 2
_NCORES = 2
_NSUB = 16
_CHUNK = 128
_NT = _NNZ // _CHUNK
_f32 = jnp.float32

_HIGH = lax.Precision.HIGHEST


def _mm_body(x_ref, w_ref, o_ref):
    o_ref[...] = lax.dot_general(
        x_ref[...], w_ref[...], (((1,), (1,)), ((), ())),
        preferred_element_type=_f32, precision=_HIGH)


def _phase_a(X, W):
    bn = 2000
    nb = _N // bn
    return pl.pallas_call(
        _mm_body,
        grid=(_NCORES, nb),
        in_specs=[
            pl.BlockSpec((bn, _HC), lambda c, i: (i, 0)),
            pl.BlockSpec((_HALF, _HC), lambda c, i: (c, 0)),
        ],
        out_specs=pl.BlockSpec((bn, _HALF), lambda c, i, _nb=nb: (c * _nb + i, 0)),
        out_shape=jax.ShapeDtypeStruct((_NCORES * _N, _HALF), _f32),
    )(X, W)


def _sub_range(s):
    start = s * 156 + jnp.minimum(s, 4)
    cnt = jnp.where(s < 4, 157, 156)
    return start, cnt


def _sliced_copy(s, total, src, dst, src_off=0, dst_off=0):
    base = (total // _NSUB) // 8 * 8
    tail = total - _NSUB * base

    def cp(r0, nrows):
        so = pl.multiple_of(src_off + r0, 8)
        do = pl.multiple_of(dst_off + r0, 8)
        pltpu.sync_copy(src.at[pl.ds(so, nrows)], dst.at[pl.ds(do, nrows)])

    cp(s * base, base)
    if tail:
        @pl.when(s == _NSUB - 1)
        def _():
            cp(_NSUB * base, tail)


def _phase_b_body(x0f, vert2, edg, z64, z16, ones16,
                  xe_out, cnt_out,
                  vbuf, ebuf, rbuf, obuf, sem, xe_sh, cnt_sh):
    c = lax.axis_index("c")
    s = lax.axis_index("s")
    _sliced_copy(s, _E, z64, xe_sh)
    _sliced_copy(s, _E, z16, cnt_sh)
    pltpu.sync_copy(ones16, obuf)
    plsc.subcore_barrier()

    start, n = _sub_range(s)

    def chunk(k, carry):
        t = start + k
        off = c * _NNZ + t * _CHUNK
        eoff = t * _CHUNK
        pltpu.sync_copy(vert2.at[pl.ds(off, _CHUNK)], vbuf)
        pltpu.sync_copy(edg.at[pl.ds(eoff, _CHUNK)], ebuf)
        pltpu.async_copy(x0f.at[vbuf], rbuf, sem).wait()
        pltpu.sync_copy(rbuf, xe_sh.at[ebuf], add=True)
        pltpu.sync_copy(obuf, cnt_sh.at[ebuf], add=True)
        return carry

    lax.fori_loop(0, n, chunk, 0)
    plsc.subcore_barrier()
    _sliced_copy(s, _E, xe_sh, xe_out, dst_off=c * _E)
    _sliced_copy(s, _E, cnt_sh, cnt_out, dst_off=c * _E)


def _phase_b(x0f, vert2, edg, z64, z16, ones16):
    mesh = plsc.VectorSubcoreMesh(core_axis_name="c", subcore_axis_name="s")
    f = pl.kernel(
        _phase_b_body,
        out_type=(jax.ShapeDtypeStruct((_NCORES * _E, _HALF), _f32),
                  jax.ShapeDtypeStruct((_NCORES * _E, 16), _f32)),
        mesh=mesh,
        scratch_types=[
            pltpu.VMEM((_CHUNK,), jnp.int32),
            pltpu.VMEM((_CHUNK,), jnp.int32),
            pltpu.VMEM((_CHUNK, _HALF), _f32),
            pltpu.VMEM((_CHUNK, 16), _f32),
            pltpu.SemaphoreType.DMA,
            pltpu.VMEM_SHARED((_E, _HALF), _f32),
            pltpu.VMEM_SHARED((_E, 16), _f32),
        ],
        compiler_params=pltpu.CompilerParams(use_tc_tiling_on_sc=False),
    )
    return f(x0f, vert2, edg, z64, z16, ones16)


def _phase_c_body(xe_ref, cnt_ref, a_ref, m2_ref, ge_ref, g16_ref):
    xs = xe_ref[...]
    cnt = cnt_ref[:, 0:1]
    xe = xs / jnp.maximum(cnt, 1.0)
    al = lax.dot_general(xe, a_ref[0], (((1,), (0,)), ((), ())),
                         preferred_element_type=_f32, precision=_HIGH)
    lr = jnp.where(al >= 0.0, al, al * 0.01)
    g = jnp.exp(lr)
    gb = lax.dot_general(g, m2_ref[...], (((1,), (0,)), ((), ())),
                         preferred_element_type=_f32, precision=_HIGH)
    ge_ref[...] = gb * xe
    g16_ref[...] = g[:, :16]


def _phase_c(xe_sum, cnt, A, M2):
    be = 2000
    nb = _E // be
    return pl.pallas_call(
        _phase_c_body,
        grid=(_NCORES, nb),
        in_specs=[
            pl.BlockSpec((be, _HALF), lambda c, i, _nb=nb: (c * _nb + i, 0)),
            pl.BlockSpec((be, 16), lambda c, i: (i, 0)),
            pl.BlockSpec((1, _HALF, _HALF), lambda c, i: (c, 0, 0)),
            pl.BlockSpec((_HALF, _HALF), lambda c, i: (0, 0)),
        ],
        out_specs=[
            pl.BlockSpec((be, _HALF), lambda c, i, _nb=nb: (c * _nb + i, 0)),
            pl.BlockSpec((be, 16), lambda c, i, _nb=nb: (c * _nb + i, 0)),
        ],
        out_shape=(jax.ShapeDtypeStruct((_NCORES * _E, _HALF), _f32),
                   jax.ShapeDtypeStruct((_NCORES * _E, 16), _f32)),
    )(xe_sum, cnt, A, M2)


def _phase_d_body(gef, g16f, vert, edg2, z64, z16,
                  xn_out, den_out,
                  vbuf, ebuf, rbuf, gbuf, sem, semg, xn_sh, den_sh):
    c = lax.axis_index("c")
    s = lax.axis_index("s")
    _sliced_copy(s, _N, z64, xn_sh)
    _sliced_copy(s, _N, z16, den_sh)
    plsc.subcore_barrier()

    start, n = _sub_range(s)

    def chunk(k, carry):
        t = start + k
        eoff = c * _NNZ + t * _CHUNK
        voff = t * _CHUNK
        pltpu.sync_copy(edg2.at[pl.ds(eoff, _CHUNK)], ebuf)
        pltpu.sync_copy(vert.at[pl.ds(voff, _CHUNK)], vbuf)
        cp1 = pltpu.async_copy(gef.at[ebuf], rbuf, sem)
        cp2 = pltpu.async_copy(g16f.at[ebuf], gbuf, semg)
        cp1.wait()
        cp2.wait()
        pltpu.sync_copy(rbuf, xn_sh.at[vbuf], add=True)
        pltpu.sync_copy(gbuf, den_sh.at[vbuf], add=True)
        return carry

    lax.fori_loop(0, n, chunk, 0)
    plsc.subcore_barrier()
    _sliced_copy(s, _N, xn_sh, xn_out, dst_off=c * _N)
    _sliced_copy(s, _N, den_sh, den_out, dst_off=c * _N)


def _phase_d(gef, g16f, vert, edg2, z64, z16):
    mesh = plsc.VectorSubcoreMesh(core_axis_name="c", subcore_axis_name="s")
    f = pl.kernel(
        _phase_d_body,
        out_type=(jax.ShapeDtypeStruct((_NCORES * _N, _HALF), _f32),
                  jax.ShapeDtypeStruct((_NCORES * _N, 16), _f32)),
        mesh=mesh,
        scratch_types=[
            pltpu.VMEM((_CHUNK,), jnp.int32),
            pltpu.VMEM((_CHUNK,), jnp.int32),
            pltpu.VMEM((_CHUNK, _HALF), _f32),
            pltpu.VMEM((_CHUNK, 16), _f32),
            pltpu.SemaphoreType.DMA,
            pltpu.SemaphoreType.DMA,
            pltpu.VMEM_SHARED((_N, _HALF), _f32),
            pltpu.VMEM_SHARED((_N, 16), _f32),
        ],
        compiler_params=pltpu.CompilerParams(use_tc_tiling_on_sc=False),
    )
    return f(gef, g16f, vert, edg2, z64, z16)


def _phase_e_body(xn0, xn1, dn0, dn1, x00, x01, msa, msb, o_ref):
    denb = (lax.dot_general(dn0[...], msa[...], (((1,), (0,)), ((), ())),
                            preferred_element_type=_f32, precision=_HIGH)
            + lax.dot_general(dn1[...], msb[...], (((1,), (0,)), ((), ())),
                              preferred_element_type=_f32, precision=_HIGH))
    num = jnp.concatenate([xn0[...], xn1[...]], axis=1)
    x0 = jnp.concatenate([x00[...], x01[...]], axis=1)
    o_ref[...] = num / (denb + 1e-16) + x0


def _phase_e(xn, den, x0f, MselA, MselB):
    bn = 2000
    nb = _N // bn

    def lo(i):
        return (i, 0)

    def hi(i, _nb=nb):
        return (_nb + i, 0)

    return pl.pallas_call(
        _phase_e_body,
        grid=(nb,),
        in_specs=[
            pl.BlockSpec((bn, _HALF), lo),
            pl.BlockSpec((bn, _HALF), hi),
            pl.BlockSpec((bn, 16), lo),
            pl.BlockSpec((bn, 16), hi),
            pl.BlockSpec((bn, _HALF), lo),
            pl.BlockSpec((bn, _HALF), hi),
            pl.BlockSpec((16, _HC), lambda i: (0, 0)),
            pl.BlockSpec((16, _HC), lambda i: (0, 0)),
        ],
        out_specs=pl.BlockSpec((bn, _HC), lo),
        out_shape=jax.ShapeDtypeStruct((_N, _HC), _f32),
    )(xn, xn, den, den, x0f, x0f, MselA, MselB)


def kernel(X, vertex, edges, W, att_e):
    x0f = _phase_a(X, W)

    vert2 = jnp.concatenate([vertex, vertex + _N])
    edg2 = jnp.concatenate([edges, edges + _E])
    z64 = jnp.zeros((_E, _HALF), _f32)
    z16 = jnp.zeros((_E, 16), _f32)
    ones16 = jnp.ones((_CHUNK, 16), _f32)

    xe_sum, cnt = _phase_b(x0f, vert2, edges, z64, z16, ones16)

    attf = att_e.reshape(_H, _C)
    A = jnp.zeros((_NCORES, _HALF, _HALF), _f32)
    A = A.at[0, 0:32, 0].set(attf[0]).at[0, 32:64, 1].set(attf[1])
    A = A.at[1, 0:32, 0].set(attf[2]).at[1, 32:64, 1].set(attf[3])
    M2 = jnp.zeros((_HALF, _HALF), _f32).at[0, 0:32].set(1.0).at[1, 32:64].set(1.0)

    ge, g16 = _phase_c(xe_sum, cnt, A, M2)

    xn, den = _phase_d(ge, g16, vertex, edg2, z64, z16)

    MselA = jnp.zeros((16, _HC), _f32).at[0, 0:32].set(1.0).at[1, 32:64].set(1.0)
    MselB = jnp.zeros((16, _HC), _f32).at[0, 64:96].set(1.0).at[1, 96:128].set(1.0)

    return _phase_e(xn, den, x0f, MselA, MselB)
